# baseline (speedup 1.0000x reference)
import jax
import jax.numpy as jnp
from jax import lax
from jax.experimental import pallas as pl
from jax.experimental.pallas import tpu as pltpu
from jax.experimental.pallas import tpu_sc as plsc

CELL_N = 4096
GENE_N = 256
N_FRAG = 1048576
D_EMB = 64
NSEG = CELL_N * GENE_N

FRAG_PER_ROW = 128
N_ROW = N_FRAG // FRAG_PER_ROW

F_BLK = 16384
P_BLK = F_BLK // 2
R_BLK = F_BLK // FRAG_PER_ROW

NC = 2
NS = 16
NW = NC * NS
BIN_PER_W = NSEG // NW
TRASH = BIN_PER_W
CHUNK = 32768


def _matvec(e_t, w2, m_mask, g_sum):
    def body(x_ref, w_ref, m_ref, g_ref, o_ref):
        y = jax.lax.dot_general(x_ref[...], w_ref[...],
                                (((0,), (0,)), ((), ())),
                                preferred_element_type=jnp.float32)
        o_ref[...] = jnp.dot(g_ref[...], y * m_ref[...],
                             preferred_element_type=jnp.float32)

    return pl.pallas_call(
        body,
        grid=(N_FRAG // F_BLK,),
        in_specs=[
            pl.BlockSpec((D_EMB, F_BLK), lambda i: (0, i)),
            pl.BlockSpec((D_EMB, FRAG_PER_ROW), lambda i: (0, 0)),
            pl.BlockSpec((F_BLK, FRAG_PER_ROW), lambda i: (0, 0)),
            pl.BlockSpec((R_BLK, F_BLK), lambda i: (0, 0)),
        ],
        out_specs=pl.BlockSpec((R_BLK, FRAG_PER_ROW), lambda i: (i, 0)),
        out_shape=jax.ShapeDtypeStruct((N_ROW, FRAG_PER_ROW), jnp.float32),
    )(e_t, w2, m_mask, g_sum)


def _scatter_body(ids_hbm, vals_hbm, bounds_hbm, out_hbm,
                  idx_v, val_v, bounds_v, acc):
    w = lax.axis_index("c") * NS + lax.axis_index("s")
    base = w * BIN_PER_W

    def zfill(i, _):
        acc[pl.ds(i * 16, 16)] = jnp.zeros((16,), jnp.float32)
        return 0

    lax.fori_loop(0, (BIN_PER_W + 16) // 16, zfill, 0)

    pltpu.sync_copy(bounds_hbm, bounds_v)
    bpair = bounds_v[pl.ds(w * 8, 16)]
    fs = bpair[0]
    fe = bpair[1]
    fs8 = pl.multiple_of(fs & ~7, 8)
    nch = (fe - fs8 + CHUNK - 1) // CHUNK

    def chunk_loop(i, _):
        off = pl.multiple_of(fs8 + i * CHUNK, 8)
        pltpu.sync_copy(ids_hbm.at[pl.ds(off, CHUNK)], idx_v)
        pltpu.sync_copy(vals_hbm.at[pl.ds(off, CHUNK)], val_v)

        def scat(j, _):
            v = idx_v[pl.ds(j * 16, 16)] - base
            ok = (v >= 0) & (v < BIN_PER_W)
            tgt = jnp.where(ok, v, jnp.full((16,), TRASH, jnp.int32))
            plsc.addupdate_scatter(acc, [tgt], val_v[pl.ds(j * 16, 16)])
            return 0

        lax.fori_loop(0, CHUNK // 16, scat, 0)
        return 0

    lax.fori_loop(0, nch, chunk_loop, 0)

    pltpu.sync_copy(acc.at[pl.ds(0, BIN_PER_W)],
                    out_hbm.at[pl.ds(base, BIN_PER_W)])


def _scatter(ids_p, vals_p, bounds):
    mesh = plsc.VectorSubcoreMesh(core_axis_name="c", subcore_axis_name="s")
    return pl.kernel(
        _scatter_body,
        mesh=mesh,
        out_type=jax.ShapeDtypeStruct((NSEG,), jnp.float32),
        compiler_params=pltpu.CompilerParams(needs_layout_passes=False),
        scratch_types=[
            pltpu.VMEM((CHUNK,), jnp.int32),
            pltpu.VMEM((CHUNK,), jnp.float32),
            pltpu.VMEM(((NW + 8) * 8,), jnp.int32),
            pltpu.VMEM((BIN_PER_W + 16,), jnp.float32),
        ],
    )(ids_p, vals_p, bounds)


def _finalize(acc128, bias_row):
    R = 256

    def body(p_ref, b_ref, o_ref):
        o_ref[...] = p_ref[...].reshape(R, GENE_N) + b_ref[...]

    return pl.pallas_call(
        body,
        grid=(CELL_N // R,),
        in_specs=[
            pl.BlockSpec((2 * R, FRAG_PER_ROW), lambda i: (i, 0)),
            pl.BlockSpec((1, GENE_N), lambda i: (0, 0)),
        ],
        out_specs=pl.BlockSpec((R, GENE_N), lambda i: (i, 0)),
        out_shape=jax.ShapeDtypeStruct((CELL_N, GENE_N), jnp.float32),
    )(acc128, bias_row)


def kernel(fragment_embedding, fragment_cellxgene_ix, cell_n, gene_n, gene_ix,
           weight1, bias1):
    offset = (cell_n * gene_n - NSEG).astype(jnp.int32)
    ids = fragment_cellxgene_ix.astype(jnp.int32) + offset
    ids = jnp.clip(ids, 0, NSEG - 1)

    edges = jnp.arange(NW + 1, dtype=jnp.int32) * BIN_PER_W
    b = jnp.searchsorted(ids, edges).astype(jnp.int32)
    bounds = jnp.pad(jnp.stack([b[:NW], b[1:NW + 1]], axis=1),
                     ((0, 8), (0, 6))).reshape(-1)
    ids_p = jnp.pad(ids, (0, CHUNK), constant_values=NSEG)

    w2 = weight1.astype(jnp.float32)[:, None] * jnp.ones(
        (1, FRAG_PER_ROW), jnp.float32)
    m_mask = jnp.tile(jnp.eye(FRAG_PER_ROW, dtype=jnp.float32),
                      (F_BLK // FRAG_PER_ROW, 1))
    g_sum = jnp.repeat(jnp.eye(R_BLK, dtype=jnp.float32), FRAG_PER_ROW,
                       axis=1)

    e_t = jnp.transpose(fragment_embedding)
    scalars = _matvec(e_t, w2, m_mask, g_sum)
    acc = _scatter(ids, scalars.reshape(N_FRAG), bounds)

    bias_row = bias1[gene_ix].astype(jnp.float32).reshape(1, GENE_N)
    return _finalize(acc.reshape(N_ROW, FRAG_PER_ROW), bias_row)


# speedup vs baseline: 1.0801x; 1.0801x over previous
import jax
import jax.numpy as jnp
from jax import lax
from jax.experimental import pallas as pl
from jax.experimental.pallas import tpu as pltpu
from jax.experimental.pallas import tpu_sc as plsc

CELL_N = 4096
GENE_N = 256
N_FRAG = 1048576
D_EMB = 64
NSEG = CELL_N * GENE_N

FRAG_PER_ROW = 128
N_ROW = N_FRAG // FRAG_PER_ROW

F_BLK = 16384
P_BLK = F_BLK // 2
R_BLK = F_BLK // FRAG_PER_ROW

NC = 2
NS = 16
NW = NC * NS
BIN_PER_W = NSEG // NW
TRASH = BIN_PER_W
CHUNK = 16384


def _matvec(e_t, w2, m_mask, g_sum):
    def body(x_ref, w_ref, m_ref, g_ref, o_ref):
        y = jax.lax.dot_general(x_ref[...], w_ref[...],
                                (((0,), (0,)), ((), ())),
                                preferred_element_type=jnp.float32)
        o_ref[...] = jnp.dot(g_ref[...], y * m_ref[...],
                             preferred_element_type=jnp.float32)

    return pl.pallas_call(
        body,
        grid=(N_FRAG // F_BLK,),
        in_specs=[
            pl.BlockSpec((D_EMB, F_BLK), lambda i: (0, i)),
            pl.BlockSpec((D_EMB, FRAG_PER_ROW), lambda i: (0, 0)),
            pl.BlockSpec((F_BLK, FRAG_PER_ROW), lambda i: (0, 0)),
            pl.BlockSpec((R_BLK, F_BLK), lambda i: (0, 0)),
        ],
        out_specs=pl.BlockSpec((R_BLK, FRAG_PER_ROW), lambda i: (i, 0)),
        out_shape=jax.ShapeDtypeStruct((N_ROW, FRAG_PER_ROW), jnp.float32),
    )(e_t, w2, m_mask, g_sum)


def _scatter_body(ids_hbm, vals_hbm, bounds_hbm, out_hbm,
                  idx_v, val_v, bounds_v, acc):
    w = lax.axis_index("c") * NS + lax.axis_index("s")
    base = w * BIN_PER_W

    def zfill(i, _):
        acc[pl.ds(i * 16, 16)] = jnp.zeros((16,), jnp.float32)
        return 0

    lax.fori_loop(0, (BIN_PER_W + 16) // 16, zfill, 0)

    pltpu.sync_copy(bounds_hbm, bounds_v)
    bpair = bounds_v[pl.ds(w * 8, 16)]
    fs = bpair[0]
    fe = bpair[1]
    fs8 = pl.multiple_of(fs & ~7, 8)
    nch = (fe - fs8 + CHUNK - 1) // CHUNK

    def chunk_loop(i, _):
        off = pl.multiple_of(fs8 + i * CHUNK, 8)
        pltpu.sync_copy(ids_hbm.at[pl.ds(off, CHUNK)], idx_v)
        pltpu.sync_copy(vals_hbm.at[pl.ds(off, CHUNK)], val_v)

        def scat(j, _):
            v = idx_v[pl.ds(j * 16, 16)] - base
            ok = (v >= 0) & (v < BIN_PER_W)
            tgt = jnp.where(ok, v, jnp.full((16,), TRASH, jnp.int32))
            plsc.addupdate_scatter(acc, [tgt], val_v[pl.ds(j * 16, 16)])
            return 0

        lax.fori_loop(0, CHUNK // 16, scat, 0)
        return 0

    lax.fori_loop(0, nch, chunk_loop, 0)

    pltpu.sync_copy(acc.at[pl.ds(0, BIN_PER_W)],
                    out_hbm.at[pl.ds(base, BIN_PER_W)])


def _scatter(ids_p, vals_p, bounds):
    mesh = plsc.VectorSubcoreMesh(core_axis_name="c", subcore_axis_name="s")
    return pl.kernel(
        _scatter_body,
        mesh=mesh,
        out_type=jax.ShapeDtypeStruct((NSEG,), jnp.float32),
        compiler_params=pltpu.CompilerParams(needs_layout_passes=False),
        scratch_types=[
            pltpu.VMEM((CHUNK,), jnp.int32),
            pltpu.VMEM((CHUNK,), jnp.float32),
            pltpu.VMEM(((NW + 8) * 8,), jnp.int32),
            pltpu.VMEM((BIN_PER_W + 16,), jnp.float32),
        ],
    )(ids_p, vals_p, bounds)


def _finalize(acc128, bias_row):
    R = 256

    def body(p_ref, b_ref, o_ref):
        o_ref[...] = p_ref[...].reshape(R, GENE_N) + b_ref[...]

    return pl.pallas_call(
        body,
        grid=(CELL_N // R,),
        in_specs=[
            pl.BlockSpec((2 * R, FRAG_PER_ROW), lambda i: (i, 0)),
            pl.BlockSpec((1, GENE_N), lambda i: (0, 0)),
        ],
        out_specs=pl.BlockSpec((R, GENE_N), lambda i: (i, 0)),
        out_shape=jax.ShapeDtypeStruct((CELL_N, GENE_N), jnp.float32),
    )(acc128, bias_row)


def kernel(fragment_embedding, fragment_cellxgene_ix, cell_n, gene_n, gene_ix,
           weight1, bias1):
    offset = (cell_n * gene_n - NSEG).astype(jnp.int32)
    ids = fragment_cellxgene_ix.astype(jnp.int32) + offset
    ids = jnp.clip(ids, 0, NSEG - 1)

    edges = jnp.arange(NW + 1, dtype=jnp.int32) * BIN_PER_W
    b = jnp.searchsorted(ids, edges).astype(jnp.int32)
    bounds = jnp.pad(jnp.stack([b[:NW], b[1:NW + 1]], axis=1),
                     ((0, 8), (0, 6))).reshape(-1)
    ids_p = jnp.pad(ids, (0, CHUNK), constant_values=NSEG)

    w2 = weight1.astype(jnp.float32)[:, None] * jnp.ones(
        (1, FRAG_PER_ROW), jnp.float32)
    m_mask = jnp.tile(jnp.eye(FRAG_PER_ROW, dtype=jnp.float32),
                      (F_BLK // FRAG_PER_ROW, 1))
    g_sum = jnp.repeat(jnp.eye(R_BLK, dtype=jnp.float32), FRAG_PER_ROW,
                       axis=1)

    e_t = jnp.transpose(fragment_embedding)
    scalars = _matvec(e_t, w2, m_mask, g_sum)
    acc = _scatter(ids, scalars.reshape(N_FRAG), bounds)

    bias_row = bias1[gene_ix].astype(jnp.float32).reshape(1, GENE_N)
    return _finalize(acc.reshape(N_ROW, FRAG_PER_ROW), bias_row)
